# baseline (speedup 1.0000x reference)
import functools

import jax
import jax.numpy as jnp
from jax import lax
from jax.experimental import pallas as pl
from jax.experimental.pallas import tpu as pltpu
from jax.experimental.pallas import tpu_sc as plsc

N = 50000
NS = 16
CH = 64
EPS = 1e-5
M = N * NS

NCHUNK = N // 8
NWORK = 32
BASE_C = NCHUNK // NWORK
EXTRA = NCHUNK - BASE_C * NWORK
MAXC = BASE_C + 1
NPAD = 50048

BR = 400
GRID = N // BR
BRS = BR * NS

_f32 = jnp.float32
_bf16 = jnp.bfloat16


def _k0_body(x_ref, wq_ref, bq_ref, wk_ref, bk_ref, wv_ref, bv_ref,
             xq_ref, xk_ref, xv_ref):
    xb = x_ref[...]
    xq_ref[...] = jnp.dot(xb, wq_ref[...], preferred_element_type=_f32) + bq_ref[...]
    xk_ref[...] = (jnp.dot(xb, wk_ref[...], preferred_element_type=_f32)
                   + bk_ref[...]).astype(_bf16)
    xv_ref[...] = (jnp.dot(xb, wv_ref[...], preferred_element_type=_f32)
                   + bv_ref[...]).astype(_bf16)


def _proj(x, Wq, bq, Wk, bk, Wv, bv):
    blk = 2000
    grid = (N // blk,)
    row_spec = pl.BlockSpec((blk, CH), lambda i: (i, 0))
    w_spec = pl.BlockSpec((CH, CH), lambda i: (0, 0))
    b_spec = pl.BlockSpec((1, CH), lambda i: (0, 0))
    return pl.pallas_call(
        _k0_body,
        grid=grid,
        in_specs=[row_spec, w_spec, b_spec, w_spec, b_spec, w_spec, b_spec],
        out_specs=[row_spec, row_spec, row_spec],
        out_shape=[jax.ShapeDtypeStruct((N, CH), _f32),
                   jax.ShapeDtypeStruct((N, CH), _bf16),
                   jax.ShapeDtypeStruct((N, CH), _bf16)],
    )(x, Wq, bq.reshape(1, CH), Wk, bk.reshape(1, CH), Wv, bv.reshape(1, CH))


def _sc_work_split(wid):
    nc = BASE_C + jnp.where(wid < EXTRA, 1, 0)
    base = wid * BASE_C + jnp.minimum(wid, EXTRA)
    return nc, base


def _sc_idx_gxyz(p_pad, edges, boundary):
    mesh = plsc.VectorSubcoreMesh(core_axis_name="c", subcore_axis_name="s")

    @functools.partial(
        pl.kernel,
        mesh=mesh,
        compiler_params=pltpu.CompilerParams(
            needs_layout_passes=False, use_tc_tiling_on_sc=False),
        out_type=[
            jax.ShapeDtypeStruct((NCHUNK, 128), jnp.int32),
            jax.ShapeDtypeStruct((M, 16), _f32),
        ],
        scratch_types=[
            pltpu.VMEM((NPAD,), jnp.int32),
            pltpu.VMEM((MAXC, 128), jnp.int32),
            pltpu.VMEM((8, NS), jnp.int32),
            pltpu.VMEM((128, 16), _f32),
            pltpu.VMEM((8, 16), _f32),
            pltpu.VMEM((128, 16), _f32),
            pltpu.SemaphoreType.DMA,
        ],
    )
    def sc_kernel(p_hbm, edges_hbm, bnd_hbm,
                  idx_out, gxyz_out,
                  bnd_v, idx_v, edg_v, pg_v, selfp_v, gx_v, semp):
        wid = lax.axis_index("s") * 2 + lax.axis_index("c")
        nc, base = _sc_work_split(wid)

        pltpu.sync_copy(bnd_hbm, bnd_v.at[pl.ds(0, N)])

        def phase1(k, carry):
            g = base + k
            pltpu.sync_copy(edges_hbm.at[pl.ds(g * 8, 8)], edg_v)
            for pt in range(8):
                i = g * 8 + pt
                erow = edg_v[pt, :]
                self_vec = jnp.full((16,), i, jnp.int32)
                nb = plsc.load_gather(bnd_v, [erow])
                sb = plsc.load_gather(bnd_v, [self_vec])
                sel = jnp.where(nb == sb, erow, self_vec)
                idx_v[k, pl.ds(pt * 16, 16)] = sel
            return carry

        lax.fori_loop(0, nc, phase1, 0)
        pltpu.sync_copy(idx_v.at[pl.ds(0, BASE_C)],
                        idx_out.at[pl.ds(base, BASE_C)])

        @pl.when(wid < EXTRA)
        def _():
            pltpu.sync_copy(idx_v.at[pl.ds(BASE_C, 1)],
                            idx_out.at[pl.ds(base + BASE_C, 1)])

        def phase2(k, carry):
            g = base + k
            cp = pltpu.async_copy(p_hbm.at[idx_v.at[k]], pg_v, semp)
            pltpu.sync_copy(p_hbm.at[pl.ds(g * 8, 8)], selfp_v)
            cp.wait()
            for pt in range(8):
                srow = selfp_v[pt, :]
                for j in range(NS):
                    r = pt * NS + j
                    gx_v[r, :] = pg_v[r, :] - srow
            pltpu.sync_copy(gx_v, gxyz_out.at[pl.ds(g * 128, 128)])
            return carry

        lax.fori_loop(0, nc, phase2, 0)

    return sc_kernel(p_pad, edges, boundary)


def _sc_kv_gather(xk, xv, idx):
    mesh = plsc.VectorSubcoreMesh(core_axis_name="c", subcore_axis_name="s")

    @functools.partial(
        pl.kernel,
        mesh=mesh,
        compiler_params=pltpu.CompilerParams(
            needs_layout_passes=False, use_tc_tiling_on_sc=False),
        out_type=[
            jax.ShapeDtypeStruct((M, CH), _bf16),
            jax.ShapeDtypeStruct((M, CH), _bf16),
        ],
        scratch_types=[
            pltpu.VMEM((MAXC, 128), jnp.int32),
            pltpu.VMEM((128, CH), _bf16),
            pltpu.VMEM((128, CH), _bf16),
            pltpu.SemaphoreType.DMA,
            pltpu.SemaphoreType.DMA,
        ],
    )
    def sc_kernel(xk_hbm, xv_hbm, idx_hbm,
                  xkg_out, xvg_out,
                  idx_v, xkr_v, xvr_v, semk, semv):
        wid = lax.axis_index("s") * 2 + lax.axis_index("c")
        nc, base = _sc_work_split(wid)
        pltpu.sync_copy(idx_hbm.at[pl.ds(base, BASE_C)],
                        idx_v.at[pl.ds(0, BASE_C)])

        @pl.when(wid < EXTRA)
        def _():
            pltpu.sync_copy(idx_hbm.at[pl.ds(base + BASE_C, 1)],
                            idx_v.at[pl.ds(BASE_C, 1)])

        def phase2(k, carry):
            g = base + k
            ck = pltpu.async_copy(xk_hbm.at[idx_v.at[k]], xkr_v, semk)
            cv = pltpu.async_copy(xv_hbm.at[idx_v.at[k]], xvr_v, semv)
            ck.wait()
            pltpu.sync_copy(xkr_v, xkg_out.at[pl.ds(g * 128, 128)])
            cv.wait()
            pltpu.sync_copy(xvr_v, xvg_out.at[pl.ds(g * 128, 128)])
            return carry

        lax.fori_loop(0, nc, phase2, 0)

    return sc_kernel(xk, xv, idx)


def _acc(o_ref, blk):
    i = pl.program_id(0)

    @pl.when(i == 0)
    def _():
        o_ref[...] = blk

    @pl.when(i > 0)
    def _():
        o_ref[...] += blk


def _pr1024(g_ref, kp1_ref, a1_ref, c1_ref, kp2_ref, pb2_ref):
    r1 = jnp.dot(g_ref[...], kp1_ref[...], preferred_element_type=_f32)
    h1 = jnp.maximum(r1 * a1_ref[...] + c1_ref[...], 0.0)
    return jnp.dot(h1, kp2_ref[...], preferred_element_type=_f32) + pb2_ref[...]


def _w1024(g_ref, xk_ref, xq_ref, kp1_ref, a1_ref, c1_ref, kp2_ref, pb2_ref,
           t2_ref):
    pr = _pr1024(g_ref, kp1_ref, a1_ref, c1_ref, kp2_ref, pb2_ref)
    xqt = jnp.dot(xq_ref[...], t2_ref[...], preferred_element_type=_f32)
    return xk_ref[...].astype(_f32) - xqt + pr


def _k1_body(g_ref, kp1_ref, pb1_ref, f16_ref, o_ref):
    r1 = jnp.dot(g_ref[...], kp1_ref[...], preferred_element_type=_f32) + pb1_ref[...]
    s = jnp.dot(jnp.sum(r1, axis=0, keepdims=True), f16_ref[...],
                preferred_element_type=_f32)
    ss = jnp.dot(jnp.sum(r1 * r1, axis=0, keepdims=True), f16_ref[...],
                 preferred_element_type=_f32)
    _acc(o_ref, jnp.concatenate([s, ss, jnp.zeros((6, 16), _f32)], axis=0))


def _bn1_stats(g256, KP1, pb1t, F16):
    return pl.pallas_call(
        _k1_body,
        grid=(GRID,),
        in_specs=[
            pl.BlockSpec((BR, 256), lambda i: (i, 0)),
            pl.BlockSpec((256, 256), lambda i: (0, 0)),
            pl.BlockSpec((1, 256), lambda i: (0, 0)),
            pl.BlockSpec((256, 16), lambda i: (0, 0)),
        ],
        out_specs=pl.BlockSpec((8, 16), lambda i: (0, 0)),
        out_shape=jax.ShapeDtypeStruct((8, 16), _f32),
    )(g256, KP1, pb1t, F16)


def _k2_body(g_ref, xk_ref, xq_ref, kp1_ref, a1_ref, c1_ref, kp2_ref, pb2_ref,
             t2_ref, gm_ref, o_ref):
    w = _w1024(g_ref, xk_ref, xq_ref, kp1_ref, a1_ref, c1_ref, kp2_ref,
               pb2_ref, t2_ref)
    s = jnp.dot(jnp.sum(w, axis=0, keepdims=True), gm_ref[...],
                preferred_element_type=_f32)
    ss = jnp.dot(jnp.sum(w * w, axis=0, keepdims=True), gm_ref[...],
                 preferred_element_type=_f32)
    _acc(o_ref, jnp.concatenate([s, ss, jnp.zeros((6, CH), _f32)], axis=0))


def _bn2_stats(g256, xk1024, xq, KP1, A1t, C1t, KP2, pb2t, T2, GM):
    return pl.pallas_call(
        _k2_body,
        grid=(GRID,),
        in_specs=[
            pl.BlockSpec((BR, 256), lambda i: (i, 0)),
            pl.BlockSpec((BR, 1024), lambda i: (i, 0)),
            pl.BlockSpec((BR, CH), lambda i: (i, 0)),
            pl.BlockSpec((256, 256), lambda i: (0, 0)),
            pl.BlockSpec((1, 256), lambda i: (0, 0)),
            pl.BlockSpec((1, 256), lambda i: (0, 0)),
            pl.BlockSpec((256, 1024), lambda i: (0, 0)),
            pl.BlockSpec((1, 1024), lambda i: (0, 0)),
            pl.BlockSpec((CH, 1024), lambda i: (0, 0)),
            pl.BlockSpec((1024, CH), lambda i: (0, 0)),
        ],
        out_specs=pl.BlockSpec((8, CH), lambda i: (0, 0)),
        out_shape=jax.ShapeDtypeStruct((8, CH), _f32),
    )(g256, xk1024, xq, KP1, A1t, C1t, KP2, pb2t, T2, GM)


def _k3_body(g_ref, xk_ref, xq_ref, kp1_ref, a1_ref, c1_ref, kp2_ref, pb2_ref,
             t2_ref, a2_ref, c2_ref, kw1_ref, wb1_ref, f8_ref, h_ref, o_ref):
    w = _w1024(g_ref, xk_ref, xq_ref, kp1_ref, a1_ref, c1_ref, kp2_ref,
               pb2_ref, t2_ref)
    h2 = jnp.maximum(w * a2_ref[...] + c2_ref[...], 0.0)
    hp = jnp.dot(h2, kw1_ref[...], preferred_element_type=_f32) + wb1_ref[...]
    h_ref[...] = hp
    s = jnp.dot(jnp.sum(hp, axis=0, keepdims=True), f8_ref[...],
                preferred_element_type=_f32)
    ss = jnp.dot(jnp.sum(hp * hp, axis=0, keepdims=True), f8_ref[...],
                 preferred_element_type=_f32)
    _acc(o_ref, jnp.concatenate([s, ss, jnp.zeros((6, 8), _f32)], axis=0))


def _k3(g256, xk1024, xq, KP1, A1t, C1t, KP2, pb2t, T2, A2t, C2t, KW1, wb1t, F8):
    return pl.pallas_call(
        _k3_body,
        grid=(GRID,),
        in_specs=[
            pl.BlockSpec((BR, 256), lambda i: (i, 0)),
            pl.BlockSpec((BR, 1024), lambda i: (i, 0)),
            pl.BlockSpec((BR, CH), lambda i: (i, 0)),
            pl.BlockSpec((256, 256), lambda i: (0, 0)),
            pl.BlockSpec((1, 256), lambda i: (0, 0)),
            pl.BlockSpec((1, 256), lambda i: (0, 0)),
            pl.BlockSpec((256, 1024), lambda i: (0, 0)),
            pl.BlockSpec((1, 1024), lambda i: (0, 0)),
            pl.BlockSpec((CH, 1024), lambda i: (0, 0)),
            pl.BlockSpec((1, 1024), lambda i: (0, 0)),
            pl.BlockSpec((1, 1024), lambda i: (0, 0)),
            pl.BlockSpec((1024, 128), lambda i: (0, 0)),
            pl.BlockSpec((1, 128), lambda i: (0, 0)),
            pl.BlockSpec((128, 8), lambda i: (0, 0)),
        ],
        out_specs=[
            pl.BlockSpec((BR, 128), lambda i: (i, 0)),
            pl.BlockSpec((8, 8), lambda i: (0, 0)),
        ],
        out_shape=[
            jax.ShapeDtypeStruct((N, 128), _f32),
            jax.ShapeDtypeStruct((8, 8), _f32),
        ],
    )(g256, xk1024, xq, KP1, A1t, C1t, KP2, pb2t, T2, A2t, C2t, KW1, wb1t, F8)


def _k4_body(h_ref, g_ref, xv_ref, kp1_ref, a1_ref, c1_ref, kp2_ref, pb2_ref,
             a3_ref, c3_ref, wblk_ref, wb2_ref, dm_ref, em_ref, gm_ref, o_ref):
    h3 = jnp.maximum(h_ref[...] * a3_ref[...] + c3_ref[...], 0.0)
    lg = jnp.dot(h3, wblk_ref[...], preferred_element_type=_f32) + wb2_ref[...]
    mx = jnp.max(lg, axis=1, keepdims=True)
    e = jnp.exp(lg - mx)
    den = jnp.dot(e, dm_ref[...], preferred_element_type=_f32)
    sm = e / den
    wt = jnp.dot(sm, em_ref[...], preferred_element_type=_f32)
    pr = _pr1024(g_ref, kp1_ref, a1_ref, c1_ref, kp2_ref, pb2_ref)
    vw = (xv_ref[...].astype(_f32) + pr) * wt
    o_ref[...] = jnp.dot(vw, gm_ref[...], preferred_element_type=_f32)


def _k4(hpre, g256, xv1024, KP1, A1t, C1t, KP2, pb2t, a3t, c3t,
        Wblk, wb2t, DM, EM, GM):
    return pl.pallas_call(
        _k4_body,
        grid=(GRID,),
        in_specs=[
            pl.BlockSpec((BR, 128), lambda i: (i, 0)),
            pl.BlockSpec((BR, 256), lambda i: (i, 0)),
            pl.BlockSpec((BR, 1024), lambda i: (i, 0)),
            pl.BlockSpec((256, 256), lambda i: (0, 0)),
            pl.BlockSpec((1, 256), lambda i: (0, 0)),
            pl.BlockSpec((1, 256), lambda i: (0, 0)),
            pl.BlockSpec((256, 1024), lambda i: (0, 0)),
            pl.BlockSpec((1, 1024), lambda i: (0, 0)),
            pl.BlockSpec((1, 128), lambda i: (0, 0)),
            pl.BlockSpec((1, 128), lambda i: (0, 0)),
            pl.BlockSpec((128, 128), lambda i: (0, 0)),
            pl.BlockSpec((1, 128), lambda i: (0, 0)),
            pl.BlockSpec((128, 128), lambda i: (0, 0)),
            pl.BlockSpec((128, 1024), lambda i: (0, 0)),
            pl.BlockSpec((1024, CH), lambda i: (0, 0)),
        ],
        out_specs=pl.BlockSpec((BR, CH), lambda i: (i, 0)),
        out_shape=jax.ShapeDtypeStruct((N, CH), _f32),
    )(hpre, g256, xv1024, KP1, A1t, C1t, KP2, pb2t, a3t, c3t,
      Wblk, wb2t, DM, EM, GM)


def kernel(p, x, o, edges, boundary, Wq, bq, Wk, bk, Wv, bv, pW1, pb1, pg,
           pbeta, pW2, pb2, wg1, wbeta1, wW1, wb1, wg2, wbeta2, wW2, wb2):
    p_pad = jnp.pad(p, ((0, 0), (0, 13)))
    edges = edges.astype(jnp.int32)
    boundary = boundary.astype(jnp.int32)

    xq, xk, xv = _proj(x, Wq, bq, Wk, bk, Wv, bv)
    idx, gxyz2 = _sc_idx_gxyz(p_pad, edges, boundary)
    xkg2, xvg2 = _sc_kv_gather(xk, xv, idx)
    g256 = gxyz2.reshape(N, 256)
    xk1024 = xkg2.reshape(N, NS * CH)
    xv1024 = xvg2.reshape(N, NS * CH)

    pW1p = jnp.zeros((16, 16), _f32).at[:3, :3].set(pW1)
    pb1p = jnp.zeros((16,), _f32).at[:3].set(pb1)
    pW2p = jnp.zeros((16, CH), _f32).at[:3].set(pW2)
    pgp = jnp.ones((16,), _f32).at[:3].set(pg)
    pbp = jnp.zeros((16,), _f32).at[:3].set(pbeta)

    I8 = jnp.eye(8, dtype=_f32)
    I16 = jnp.eye(16, dtype=_f32)
    I64 = jnp.eye(CH, dtype=_f32)
    KP1 = jnp.kron(I16, pW1p)
    KP2 = jnp.kron(I16, pW2p)
    T2 = jnp.kron(jnp.ones((1, 16), _f32), I64)
    GM = jnp.kron(jnp.ones((16, 1), _f32), I64)
    F8 = jnp.kron(jnp.ones((16, 1), _f32), I8)
    F16 = jnp.kron(jnp.ones((16, 1), _f32), I16)
    KW1 = jnp.kron(I16, wW1)
    Wblk = jnp.kron(I16, wW2)
    DM = jnp.kron(jnp.ones((16, 16), _f32), I8)
    EM = jnp.kron(I16, jnp.kron(jnp.ones((1, 8), _f32), I8))
    pb1t = jnp.tile(pb1p, 16).reshape(1, 256)

    st1 = _bn1_stats(g256, KP1, pb1t, F16)
    m1 = st1[0] / M
    v1 = st1[1] / M - m1 * m1
    a1 = pgp / jnp.sqrt(v1 + EPS)
    c1 = pbp - m1 * a1
    C1 = c1 + a1 * pb1p
    A1t = jnp.tile(a1, 16).reshape(1, 256)
    C1t = jnp.tile(C1, 16).reshape(1, 256)
    pb2t = jnp.tile(pb2, 16).reshape(1, 1024)

    st2 = _bn2_stats(g256, xk1024, xq, KP1, A1t, C1t, KP2, pb2t, T2, GM)
    m2 = st2[0] / M
    v2 = st2[1] / M - m2 * m2
    a2 = wg1 / jnp.sqrt(v2 + EPS)
    c2 = wbeta1 - m2 * a2
    A2t = jnp.tile(a2, 16).reshape(1, 1024)
    C2t = jnp.tile(c2, 16).reshape(1, 1024)

    hpre, st3 = _k3(g256, xk1024, xq, KP1, A1t, C1t, KP2, pb2t, T2,
                    A2t, C2t, KW1, jnp.tile(wb1, 16).reshape(1, 128), F8)
    m3 = st3[0] / M
    v3 = st3[1] / M - m3 * m3
    a3 = wg2 / jnp.sqrt(v3 + EPS)
    c3 = wbeta2 - m3 * a3
    a3t = jnp.tile(a3, 16).reshape(1, 128)
    c3t = jnp.tile(c3, 16).reshape(1, 128)
    wb2t = jnp.tile(wb2, 16).reshape(1, 128)

    return _k4(hpre, g256, xv1024, KP1, A1t, C1t, KP2, pb2t, a3t, c3t,
               Wblk, wb2t, DM, EM, GM)


# speedup vs baseline: 1.0603x; 1.0603x over previous
import functools

import jax
import jax.numpy as jnp
from jax import lax
from jax.experimental import pallas as pl
from jax.experimental.pallas import tpu as pltpu
from jax.experimental.pallas import tpu_sc as plsc

N = 50000
NS = 16
CH = 64
EPS = 1e-5
M = N * NS

NCHUNK = N // 8
NWORK = 32
BASE_C = NCHUNK // NWORK
EXTRA = NCHUNK - BASE_C * NWORK
MAXC = BASE_C + 1
NPAD = 50048

BR = 1000
GRID = N // BR
BRS = BR * NS

_f32 = jnp.float32
_bf16 = jnp.bfloat16


def _k0_body(x_ref, wq_ref, bq_ref, wk_ref, bk_ref, wv_ref, bv_ref,
             xq_ref, xk_ref, xv_ref):
    xb = x_ref[...]
    xq_ref[...] = jnp.dot(xb, wq_ref[...], preferred_element_type=_f32) + bq_ref[...]
    xk_ref[...] = (jnp.dot(xb, wk_ref[...], preferred_element_type=_f32)
                   + bk_ref[...]).astype(_bf16)
    xv_ref[...] = (jnp.dot(xb, wv_ref[...], preferred_element_type=_f32)
                   + bv_ref[...]).astype(_bf16)


def _proj(x, Wq, bq, Wk, bk, Wv, bv):
    blk = 2000
    grid = (N // blk,)
    row_spec = pl.BlockSpec((blk, CH), lambda i: (i, 0))
    w_spec = pl.BlockSpec((CH, CH), lambda i: (0, 0))
    b_spec = pl.BlockSpec((1, CH), lambda i: (0, 0))
    return pl.pallas_call(
        _k0_body,
        grid=grid,
        in_specs=[row_spec, w_spec, b_spec, w_spec, b_spec, w_spec, b_spec],
        out_specs=[row_spec, row_spec, row_spec],
        out_shape=[jax.ShapeDtypeStruct((N, CH), _f32),
                   jax.ShapeDtypeStruct((N, CH), _bf16),
                   jax.ShapeDtypeStruct((N, CH), _bf16)],
    )(x, Wq, bq.reshape(1, CH), Wk, bk.reshape(1, CH), Wv, bv.reshape(1, CH))


def _sc_work_split(wid):
    nc = BASE_C + jnp.where(wid < EXTRA, 1, 0)
    base = wid * BASE_C + jnp.minimum(wid, EXTRA)
    return nc, base


def _sc_idx_gxyz(p_pad, edges, boundary):
    mesh = plsc.VectorSubcoreMesh(core_axis_name="c", subcore_axis_name="s")

    @functools.partial(
        pl.kernel,
        mesh=mesh,
        compiler_params=pltpu.CompilerParams(
            needs_layout_passes=False, use_tc_tiling_on_sc=False),
        out_type=[
            jax.ShapeDtypeStruct((NCHUNK, 128), jnp.int32),
            jax.ShapeDtypeStruct((M, 16), _f32),
        ],
        scratch_types=[
            pltpu.VMEM((NPAD,), jnp.int32),
            pltpu.VMEM((MAXC, 128), jnp.int32),
            pltpu.VMEM((8, NS), jnp.int32),
            pltpu.VMEM((128, 16), _f32),
            pltpu.VMEM((8, 16), _f32),
            pltpu.VMEM((128, 16), _f32),
            pltpu.SemaphoreType.DMA,
        ],
    )
    def sc_kernel(p_hbm, edges_hbm, bnd_hbm,
                  idx_out, gxyz_out,
                  bnd_v, idx_v, edg_v, pg_v, selfp_v, gx_v, semp):
        wid = lax.axis_index("s") * 2 + lax.axis_index("c")
        nc, base = _sc_work_split(wid)

        pltpu.sync_copy(bnd_hbm, bnd_v.at[pl.ds(0, N)])

        def phase1(k, carry):
            g = base + k
            pltpu.sync_copy(edges_hbm.at[pl.ds(g * 8, 8)], edg_v)
            for pt in range(8):
                i = g * 8 + pt
                erow = edg_v[pt, :]
                self_vec = jnp.full((16,), i, jnp.int32)
                nb = plsc.load_gather(bnd_v, [erow])
                sb = plsc.load_gather(bnd_v, [self_vec])
                sel = jnp.where(nb == sb, erow, self_vec)
                idx_v[k, pl.ds(pt * 16, 16)] = sel
            return carry

        lax.fori_loop(0, nc, phase1, 0)
        pltpu.sync_copy(idx_v.at[pl.ds(0, BASE_C)],
                        idx_out.at[pl.ds(base, BASE_C)])

        @pl.when(wid < EXTRA)
        def _():
            pltpu.sync_copy(idx_v.at[pl.ds(BASE_C, 1)],
                            idx_out.at[pl.ds(base + BASE_C, 1)])

        def phase2(k, carry):
            g = base + k
            cp = pltpu.async_copy(p_hbm.at[idx_v.at[k]], pg_v, semp)
            pltpu.sync_copy(p_hbm.at[pl.ds(g * 8, 8)], selfp_v)
            cp.wait()
            for pt in range(8):
                srow = selfp_v[pt, :]
                for j in range(NS):
                    r = pt * NS + j
                    gx_v[r, :] = pg_v[r, :] - srow
            pltpu.sync_copy(gx_v, gxyz_out.at[pl.ds(g * 128, 128)])
            return carry

        lax.fori_loop(0, nc, phase2, 0)

    return sc_kernel(p_pad, edges, boundary)


def _sc_kv_gather(xk, xv, idx):
    mesh = plsc.VectorSubcoreMesh(core_axis_name="c", subcore_axis_name="s")

    @functools.partial(
        pl.kernel,
        mesh=mesh,
        compiler_params=pltpu.CompilerParams(
            needs_layout_passes=False, use_tc_tiling_on_sc=False),
        out_type=[
            jax.ShapeDtypeStruct((M, CH), _bf16),
            jax.ShapeDtypeStruct((M, CH), _bf16),
        ],
        scratch_types=[
            pltpu.VMEM((MAXC, 128), jnp.int32),
            pltpu.VMEM((128, CH), _bf16),
            pltpu.VMEM((128, CH), _bf16),
            pltpu.SemaphoreType.DMA,
            pltpu.SemaphoreType.DMA,
        ],
    )
    def sc_kernel(xk_hbm, xv_hbm, idx_hbm,
                  xkg_out, xvg_out,
                  idx_v, xkr_v, xvr_v, semk, semv):
        wid = lax.axis_index("s") * 2 + lax.axis_index("c")
        nc, base = _sc_work_split(wid)
        pltpu.sync_copy(idx_hbm.at[pl.ds(base, BASE_C)],
                        idx_v.at[pl.ds(0, BASE_C)])

        @pl.when(wid < EXTRA)
        def _():
            pltpu.sync_copy(idx_hbm.at[pl.ds(base + BASE_C, 1)],
                            idx_v.at[pl.ds(BASE_C, 1)])

        def phase2(k, carry):
            g = base + k
            ck = pltpu.async_copy(xk_hbm.at[idx_v.at[k]], xkr_v, semk)
            cv = pltpu.async_copy(xv_hbm.at[idx_v.at[k]], xvr_v, semv)
            ck.wait()
            pltpu.sync_copy(xkr_v, xkg_out.at[pl.ds(g * 128, 128)])
            cv.wait()
            pltpu.sync_copy(xvr_v, xvg_out.at[pl.ds(g * 128, 128)])
            return carry

        lax.fori_loop(0, nc, phase2, 0)

    return sc_kernel(xk, xv, idx)


def _acc(o_ref, blk):
    i = pl.program_id(0)

    @pl.when(i == 0)
    def _():
        o_ref[...] = blk

    @pl.when(i > 0)
    def _():
        o_ref[...] += blk


def _pr1024(g_ref, kp1_ref, a1_ref, c1_ref, kp2_ref, pb2_ref):
    r1 = jnp.dot(g_ref[...], kp1_ref[...], preferred_element_type=_f32)
    h1 = jnp.maximum(r1 * a1_ref[...] + c1_ref[...], 0.0)
    return jnp.dot(h1, kp2_ref[...], preferred_element_type=_f32) + pb2_ref[...]


def _w1024(g_ref, xk_ref, xq_ref, kp1_ref, a1_ref, c1_ref, kp2_ref, pb2_ref,
           t2_ref):
    pr = _pr1024(g_ref, kp1_ref, a1_ref, c1_ref, kp2_ref, pb2_ref)
    xqt = jnp.dot(xq_ref[...], t2_ref[...], preferred_element_type=_f32)
    return xk_ref[...].astype(_f32) - xqt + pr


def _k1_body(g_ref, kp1_ref, pb1_ref, f16_ref, o_ref):
    r1 = jnp.dot(g_ref[...], kp1_ref[...], preferred_element_type=_f32) + pb1_ref[...]
    s = jnp.dot(jnp.sum(r1, axis=0, keepdims=True), f16_ref[...],
                preferred_element_type=_f32)
    ss = jnp.dot(jnp.sum(r1 * r1, axis=0, keepdims=True), f16_ref[...],
                 preferred_element_type=_f32)
    _acc(o_ref, jnp.concatenate([s, ss, jnp.zeros((6, 16), _f32)], axis=0))


def _bn1_stats(g256, KP1, pb1t, F16):
    return pl.pallas_call(
        _k1_body,
        grid=(GRID,),
        in_specs=[
            pl.BlockSpec((BR, 256), lambda i: (i, 0)),
            pl.BlockSpec((256, 256), lambda i: (0, 0)),
            pl.BlockSpec((1, 256), lambda i: (0, 0)),
            pl.BlockSpec((256, 16), lambda i: (0, 0)),
        ],
        out_specs=pl.BlockSpec((8, 16), lambda i: (0, 0)),
        out_shape=jax.ShapeDtypeStruct((8, 16), _f32),
    )(g256, KP1, pb1t, F16)


def _k2_body(g_ref, xk_ref, xq_ref, kp1_ref, a1_ref, c1_ref, kp2_ref, pb2_ref,
             t2_ref, gm_ref, o_ref):
    w = _w1024(g_ref, xk_ref, xq_ref, kp1_ref, a1_ref, c1_ref, kp2_ref,
               pb2_ref, t2_ref)
    s = jnp.dot(jnp.sum(w, axis=0, keepdims=True), gm_ref[...],
                preferred_element_type=_f32)
    ss = jnp.dot(jnp.sum(w * w, axis=0, keepdims=True), gm_ref[...],
                 preferred_element_type=_f32)
    _acc(o_ref, jnp.concatenate([s, ss, jnp.zeros((6, CH), _f32)], axis=0))


def _bn2_stats(g256, xk1024, xq, KP1, A1t, C1t, KP2, pb2t, T2, GM):
    return pl.pallas_call(
        _k2_body,
        grid=(GRID,),
        in_specs=[
            pl.BlockSpec((BR, 256), lambda i: (i, 0)),
            pl.BlockSpec((BR, 1024), lambda i: (i, 0)),
            pl.BlockSpec((BR, CH), lambda i: (i, 0)),
            pl.BlockSpec((256, 256), lambda i: (0, 0)),
            pl.BlockSpec((1, 256), lambda i: (0, 0)),
            pl.BlockSpec((1, 256), lambda i: (0, 0)),
            pl.BlockSpec((256, 1024), lambda i: (0, 0)),
            pl.BlockSpec((1, 1024), lambda i: (0, 0)),
            pl.BlockSpec((CH, 1024), lambda i: (0, 0)),
            pl.BlockSpec((1024, CH), lambda i: (0, 0)),
        ],
        out_specs=pl.BlockSpec((8, CH), lambda i: (0, 0)),
        out_shape=jax.ShapeDtypeStruct((8, CH), _f32),
    )(g256, xk1024, xq, KP1, A1t, C1t, KP2, pb2t, T2, GM)


def _k3_body(g_ref, xk_ref, xq_ref, kp1_ref, a1_ref, c1_ref, kp2_ref, pb2_ref,
             t2_ref, a2_ref, c2_ref, kw1_ref, wb1_ref, f8_ref, h_ref, o_ref):
    w = _w1024(g_ref, xk_ref, xq_ref, kp1_ref, a1_ref, c1_ref, kp2_ref,
               pb2_ref, t2_ref)
    h2 = jnp.maximum(w * a2_ref[...] + c2_ref[...], 0.0)
    hp = jnp.dot(h2, kw1_ref[...], preferred_element_type=_f32) + wb1_ref[...]
    h_ref[...] = hp
    s = jnp.dot(jnp.sum(hp, axis=0, keepdims=True), f8_ref[...],
                preferred_element_type=_f32)
    ss = jnp.dot(jnp.sum(hp * hp, axis=0, keepdims=True), f8_ref[...],
                 preferred_element_type=_f32)
    _acc(o_ref, jnp.concatenate([s, ss, jnp.zeros((6, 8), _f32)], axis=0))


def _k3(g256, xk1024, xq, KP1, A1t, C1t, KP2, pb2t, T2, A2t, C2t, KW1, wb1t, F8):
    return pl.pallas_call(
        _k3_body,
        grid=(GRID,),
        in_specs=[
            pl.BlockSpec((BR, 256), lambda i: (i, 0)),
            pl.BlockSpec((BR, 1024), lambda i: (i, 0)),
            pl.BlockSpec((BR, CH), lambda i: (i, 0)),
            pl.BlockSpec((256, 256), lambda i: (0, 0)),
            pl.BlockSpec((1, 256), lambda i: (0, 0)),
            pl.BlockSpec((1, 256), lambda i: (0, 0)),
            pl.BlockSpec((256, 1024), lambda i: (0, 0)),
            pl.BlockSpec((1, 1024), lambda i: (0, 0)),
            pl.BlockSpec((CH, 1024), lambda i: (0, 0)),
            pl.BlockSpec((1, 1024), lambda i: (0, 0)),
            pl.BlockSpec((1, 1024), lambda i: (0, 0)),
            pl.BlockSpec((1024, 128), lambda i: (0, 0)),
            pl.BlockSpec((1, 128), lambda i: (0, 0)),
            pl.BlockSpec((128, 8), lambda i: (0, 0)),
        ],
        out_specs=[
            pl.BlockSpec((BR, 128), lambda i: (i, 0)),
            pl.BlockSpec((8, 8), lambda i: (0, 0)),
        ],
        out_shape=[
            jax.ShapeDtypeStruct((N, 128), _f32),
            jax.ShapeDtypeStruct((8, 8), _f32),
        ],
    )(g256, xk1024, xq, KP1, A1t, C1t, KP2, pb2t, T2, A2t, C2t, KW1, wb1t, F8)


def _k4_body(h_ref, g_ref, xv_ref, kp1_ref, a1_ref, c1_ref, kp2_ref, pb2_ref,
             a3_ref, c3_ref, wblk_ref, wb2_ref, dm_ref, em_ref, gm_ref, o_ref):
    h3 = jnp.maximum(h_ref[...] * a3_ref[...] + c3_ref[...], 0.0)
    lg = jnp.dot(h3, wblk_ref[...], preferred_element_type=_f32) + wb2_ref[...]
    mx = jnp.max(lg, axis=1, keepdims=True)
    e = jnp.exp(lg - mx)
    den = jnp.dot(e, dm_ref[...], preferred_element_type=_f32)
    sm = e / den
    wt = jnp.dot(sm, em_ref[...], preferred_element_type=_f32)
    pr = _pr1024(g_ref, kp1_ref, a1_ref, c1_ref, kp2_ref, pb2_ref)
    vw = (xv_ref[...].astype(_f32) + pr) * wt
    o_ref[...] = jnp.dot(vw, gm_ref[...], preferred_element_type=_f32)


def _k4(hpre, g256, xv1024, KP1, A1t, C1t, KP2, pb2t, a3t, c3t,
        Wblk, wb2t, DM, EM, GM):
    return pl.pallas_call(
        _k4_body,
        grid=(GRID,),
        in_specs=[
            pl.BlockSpec((BR, 128), lambda i: (i, 0)),
            pl.BlockSpec((BR, 256), lambda i: (i, 0)),
            pl.BlockSpec((BR, 1024), lambda i: (i, 0)),
            pl.BlockSpec((256, 256), lambda i: (0, 0)),
            pl.BlockSpec((1, 256), lambda i: (0, 0)),
            pl.BlockSpec((1, 256), lambda i: (0, 0)),
            pl.BlockSpec((256, 1024), lambda i: (0, 0)),
            pl.BlockSpec((1, 1024), lambda i: (0, 0)),
            pl.BlockSpec((1, 128), lambda i: (0, 0)),
            pl.BlockSpec((1, 128), lambda i: (0, 0)),
            pl.BlockSpec((128, 128), lambda i: (0, 0)),
            pl.BlockSpec((1, 128), lambda i: (0, 0)),
            pl.BlockSpec((128, 128), lambda i: (0, 0)),
            pl.BlockSpec((128, 1024), lambda i: (0, 0)),
            pl.BlockSpec((1024, CH), lambda i: (0, 0)),
        ],
        out_specs=pl.BlockSpec((BR, CH), lambda i: (i, 0)),
        out_shape=jax.ShapeDtypeStruct((N, CH), _f32),
    )(hpre, g256, xv1024, KP1, A1t, C1t, KP2, pb2t, a3t, c3t,
      Wblk, wb2t, DM, EM, GM)


def kernel(p, x, o, edges, boundary, Wq, bq, Wk, bk, Wv, bv, pW1, pb1, pg,
           pbeta, pW2, pb2, wg1, wbeta1, wW1, wb1, wg2, wbeta2, wW2, wb2):
    p_pad = jnp.pad(p, ((0, 0), (0, 13)))
    edges = edges.astype(jnp.int32)
    boundary = boundary.astype(jnp.int32)

    xq, xk, xv = _proj(x, Wq, bq, Wk, bk, Wv, bv)
    idx, gxyz2 = _sc_idx_gxyz(p_pad, edges, boundary)
    xkg2, xvg2 = _sc_kv_gather(xk, xv, idx)
    g256 = gxyz2.reshape(N, 256)
    xk1024 = xkg2.reshape(N, NS * CH)
    xv1024 = xvg2.reshape(N, NS * CH)

    pW1p = jnp.zeros((16, 16), _f32).at[:3, :3].set(pW1)
    pb1p = jnp.zeros((16,), _f32).at[:3].set(pb1)
    pW2p = jnp.zeros((16, CH), _f32).at[:3].set(pW2)
    pgp = jnp.ones((16,), _f32).at[:3].set(pg)
    pbp = jnp.zeros((16,), _f32).at[:3].set(pbeta)

    I8 = jnp.eye(8, dtype=_f32)
    I16 = jnp.eye(16, dtype=_f32)
    I64 = jnp.eye(CH, dtype=_f32)
    KP1 = jnp.kron(I16, pW1p)
    KP2 = jnp.kron(I16, pW2p)
    T2 = jnp.kron(jnp.ones((1, 16), _f32), I64)
    GM = jnp.kron(jnp.ones((16, 1), _f32), I64)
    F8 = jnp.kron(jnp.ones((16, 1), _f32), I8)
    F16 = jnp.kron(jnp.ones((16, 1), _f32), I16)
    KW1 = jnp.kron(I16, wW1)
    Wblk = jnp.kron(I16, wW2)
    DM = jnp.kron(jnp.ones((16, 16), _f32), I8)
    EM = jnp.kron(I16, jnp.kron(jnp.ones((1, 8), _f32), I8))
    pb1t = jnp.tile(pb1p, 16).reshape(1, 256)

    st1 = _bn1_stats(g256, KP1, pb1t, F16)
    m1 = st1[0] / M
    v1 = st1[1] / M - m1 * m1
    a1 = pgp / jnp.sqrt(v1 + EPS)
    c1 = pbp - m1 * a1
    C1 = c1 + a1 * pb1p
    A1t = jnp.tile(a1, 16).reshape(1, 256)
    C1t = jnp.tile(C1, 16).reshape(1, 256)
    pb2t = jnp.tile(pb2, 16).reshape(1, 1024)

    st2 = _bn2_stats(g256, xk1024, xq, KP1, A1t, C1t, KP2, pb2t, T2, GM)
    m2 = st2[0] / M
    v2 = st2[1] / M - m2 * m2
    a2 = wg1 / jnp.sqrt(v2 + EPS)
    c2 = wbeta1 - m2 * a2
    A2t = jnp.tile(a2, 16).reshape(1, 1024)
    C2t = jnp.tile(c2, 16).reshape(1, 1024)

    hpre, st3 = _k3(g256, xk1024, xq, KP1, A1t, C1t, KP2, pb2t, T2,
                    A2t, C2t, KW1, jnp.tile(wb1, 16).reshape(1, 128), F8)
    m3 = st3[0] / M
    v3 = st3[1] / M - m3 * m3
    a3 = wg2 / jnp.sqrt(v3 + EPS)
    c3 = wbeta2 - m3 * a3
    a3t = jnp.tile(a3, 16).reshape(1, 128)
    c3t = jnp.tile(c3, 16).reshape(1, 128)
    wb2t = jnp.tile(wb2, 16).reshape(1, 128)

    return _k4(hpre, g256, xv1024, KP1, A1t, C1t, KP2, pb2t, a3t, c3t,
               Wblk, wb2t, DM, EM, GM)


# speedup vs baseline: 1.0778x; 1.0165x over previous
import functools

import jax
import jax.numpy as jnp
from jax import lax
from jax.experimental import pallas as pl
from jax.experimental.pallas import tpu as pltpu
from jax.experimental.pallas import tpu_sc as plsc

N = 50000
NS = 16
CH = 64
EPS = 1e-5
M = N * NS

NCHUNK = N // 8
NWORK = 32
BASE_C = NCHUNK // NWORK
EXTRA = NCHUNK - BASE_C * NWORK
MAXC = BASE_C + 1
NPAD = 50048

BR = 2000
GRID = N // BR
BRS = BR * NS

_f32 = jnp.float32
_bf16 = jnp.bfloat16


def _k0_body(x_ref, wq_ref, bq_ref, wk_ref, bk_ref, wv_ref, bv_ref,
             xq_ref, xk_ref, xv_ref):
    xb = x_ref[...]
    xq_ref[...] = jnp.dot(xb, wq_ref[...], preferred_element_type=_f32) + bq_ref[...]
    xk_ref[...] = (jnp.dot(xb, wk_ref[...], preferred_element_type=_f32)
                   + bk_ref[...]).astype(_bf16)
    xv_ref[...] = (jnp.dot(xb, wv_ref[...], preferred_element_type=_f32)
                   + bv_ref[...]).astype(_bf16)


def _proj(x, Wq, bq, Wk, bk, Wv, bv):
    blk = 2000
    grid = (N // blk,)
    row_spec = pl.BlockSpec((blk, CH), lambda i: (i, 0))
    w_spec = pl.BlockSpec((CH, CH), lambda i: (0, 0))
    b_spec = pl.BlockSpec((1, CH), lambda i: (0, 0))
    return pl.pallas_call(
        _k0_body,
        grid=grid,
        in_specs=[row_spec, w_spec, b_spec, w_spec, b_spec, w_spec, b_spec],
        out_specs=[row_spec, row_spec, row_spec],
        out_shape=[jax.ShapeDtypeStruct((N, CH), _f32),
                   jax.ShapeDtypeStruct((N, CH), _bf16),
                   jax.ShapeDtypeStruct((N, CH), _bf16)],
    )(x, Wq, bq.reshape(1, CH), Wk, bk.reshape(1, CH), Wv, bv.reshape(1, CH))


def _sc_work_split(wid):
    nc = BASE_C + jnp.where(wid < EXTRA, 1, 0)
    base = wid * BASE_C + jnp.minimum(wid, EXTRA)
    return nc, base


def _sc_idx_gxyz(p_pad, edges, boundary):
    mesh = plsc.VectorSubcoreMesh(core_axis_name="c", subcore_axis_name="s")

    @functools.partial(
        pl.kernel,
        mesh=mesh,
        compiler_params=pltpu.CompilerParams(
            needs_layout_passes=False, use_tc_tiling_on_sc=False),
        out_type=[
            jax.ShapeDtypeStruct((NCHUNK, 128), jnp.int32),
            jax.ShapeDtypeStruct((M, 16), _f32),
        ],
        scratch_types=[
            pltpu.VMEM((NPAD,), jnp.int32),
            pltpu.VMEM((MAXC, 128), jnp.int32),
            pltpu.VMEM((8, NS), jnp.int32),
            pltpu.VMEM((128, 16), _f32),
            pltpu.VMEM((8, 16), _f32),
            pltpu.VMEM((128, 16), _f32),
            pltpu.SemaphoreType.DMA,
        ],
    )
    def sc_kernel(p_hbm, edges_hbm, bnd_hbm,
                  idx_out, gxyz_out,
                  bnd_v, idx_v, edg_v, pg_v, selfp_v, gx_v, semp):
        wid = lax.axis_index("s") * 2 + lax.axis_index("c")
        nc, base = _sc_work_split(wid)

        pltpu.sync_copy(bnd_hbm, bnd_v.at[pl.ds(0, N)])

        def phase1(k, carry):
            g = base + k
            pltpu.sync_copy(edges_hbm.at[pl.ds(g * 8, 8)], edg_v)
            for pt in range(8):
                i = g * 8 + pt
                erow = edg_v[pt, :]
                self_vec = jnp.full((16,), i, jnp.int32)
                nb = plsc.load_gather(bnd_v, [erow])
                sb = plsc.load_gather(bnd_v, [self_vec])
                sel = jnp.where(nb == sb, erow, self_vec)
                idx_v[k, pl.ds(pt * 16, 16)] = sel
            return carry

        lax.fori_loop(0, nc, phase1, 0)
        pltpu.sync_copy(idx_v.at[pl.ds(0, BASE_C)],
                        idx_out.at[pl.ds(base, BASE_C)])

        @pl.when(wid < EXTRA)
        def _():
            pltpu.sync_copy(idx_v.at[pl.ds(BASE_C, 1)],
                            idx_out.at[pl.ds(base + BASE_C, 1)])

        def phase2(k, carry):
            g = base + k
            cp = pltpu.async_copy(p_hbm.at[idx_v.at[k]], pg_v, semp)
            pltpu.sync_copy(p_hbm.at[pl.ds(g * 8, 8)], selfp_v)
            cp.wait()
            for pt in range(8):
                srow = selfp_v[pt, :]
                for j in range(NS):
                    r = pt * NS + j
                    gx_v[r, :] = pg_v[r, :] - srow
            pltpu.sync_copy(gx_v, gxyz_out.at[pl.ds(g * 128, 128)])
            return carry

        lax.fori_loop(0, nc, phase2, 0)

    return sc_kernel(p_pad, edges, boundary)


def _sc_kv_gather(xk, xv, idx):
    mesh = plsc.VectorSubcoreMesh(core_axis_name="c", subcore_axis_name="s")

    @functools.partial(
        pl.kernel,
        mesh=mesh,
        compiler_params=pltpu.CompilerParams(
            needs_layout_passes=False, use_tc_tiling_on_sc=False),
        out_type=[
            jax.ShapeDtypeStruct((M, CH), _bf16),
            jax.ShapeDtypeStruct((M, CH), _bf16),
        ],
        scratch_types=[
            pltpu.VMEM((MAXC, 128), jnp.int32),
            pltpu.VMEM((128, CH), _bf16),
            pltpu.VMEM((128, CH), _bf16),
            pltpu.SemaphoreType.DMA,
            pltpu.SemaphoreType.DMA,
        ],
    )
    def sc_kernel(xk_hbm, xv_hbm, idx_hbm,
                  xkg_out, xvg_out,
                  idx_v, xkr_v, xvr_v, semk, semv):
        wid = lax.axis_index("s") * 2 + lax.axis_index("c")
        nc, base = _sc_work_split(wid)
        pltpu.sync_copy(idx_hbm.at[pl.ds(base, BASE_C)],
                        idx_v.at[pl.ds(0, BASE_C)])

        @pl.when(wid < EXTRA)
        def _():
            pltpu.sync_copy(idx_hbm.at[pl.ds(base + BASE_C, 1)],
                            idx_v.at[pl.ds(BASE_C, 1)])

        def phase2(k, carry):
            g = base + k
            ck = pltpu.async_copy(xk_hbm.at[idx_v.at[k]], xkr_v, semk)
            cv = pltpu.async_copy(xv_hbm.at[idx_v.at[k]], xvr_v, semv)
            ck.wait()
            pltpu.sync_copy(xkr_v, xkg_out.at[pl.ds(g * 128, 128)])
            cv.wait()
            pltpu.sync_copy(xvr_v, xvg_out.at[pl.ds(g * 128, 128)])
            return carry

        lax.fori_loop(0, nc, phase2, 0)

    return sc_kernel(xk, xv, idx)


def _acc(o_ref, blk):
    i = pl.program_id(0)

    @pl.when(i == 0)
    def _():
        o_ref[...] = blk

    @pl.when(i > 0)
    def _():
        o_ref[...] += blk


def _pr1024(g_ref, kp1_ref, a1_ref, c1_ref, kp2_ref, pb2_ref):
    r1 = jnp.dot(g_ref[...], kp1_ref[...], preferred_element_type=_f32)
    h1 = jnp.maximum(r1 * a1_ref[...] + c1_ref[...], 0.0)
    return jnp.dot(h1, kp2_ref[...], preferred_element_type=_f32) + pb2_ref[...]


def _w1024(g_ref, xk_ref, xq_ref, kp1_ref, a1_ref, c1_ref, kp2_ref, pb2_ref,
           t2_ref):
    pr = _pr1024(g_ref, kp1_ref, a1_ref, c1_ref, kp2_ref, pb2_ref)
    xqt = jnp.dot(xq_ref[...], t2_ref[...], preferred_element_type=_f32)
    return xk_ref[...].astype(_f32) - xqt + pr


def _k1_body(g_ref, kp1_ref, pb1_ref, f16_ref, o_ref):
    r1 = jnp.dot(g_ref[...], kp1_ref[...], preferred_element_type=_f32) + pb1_ref[...]
    s = jnp.dot(jnp.sum(r1, axis=0, keepdims=True), f16_ref[...],
                preferred_element_type=_f32)
    ss = jnp.dot(jnp.sum(r1 * r1, axis=0, keepdims=True), f16_ref[...],
                 preferred_element_type=_f32)
    _acc(o_ref, jnp.concatenate([s, ss, jnp.zeros((6, 16), _f32)], axis=0))


def _bn1_stats(g256, KP1, pb1t, F16):
    return pl.pallas_call(
        _k1_body,
        grid=(GRID,),
        in_specs=[
            pl.BlockSpec((BR, 256), lambda i: (i, 0)),
            pl.BlockSpec((256, 256), lambda i: (0, 0)),
            pl.BlockSpec((1, 256), lambda i: (0, 0)),
            pl.BlockSpec((256, 16), lambda i: (0, 0)),
        ],
        out_specs=pl.BlockSpec((8, 16), lambda i: (0, 0)),
        out_shape=jax.ShapeDtypeStruct((8, 16), _f32),
    )(g256, KP1, pb1t, F16)


def _k2_body(g_ref, xk_ref, xq_ref, kp1_ref, a1_ref, c1_ref, kp2_ref, pb2_ref,
             t2_ref, gm_ref, o_ref):
    w = _w1024(g_ref, xk_ref, xq_ref, kp1_ref, a1_ref, c1_ref, kp2_ref,
               pb2_ref, t2_ref)
    s = jnp.dot(jnp.sum(w, axis=0, keepdims=True), gm_ref[...],
                preferred_element_type=_f32)
    ss = jnp.dot(jnp.sum(w * w, axis=0, keepdims=True), gm_ref[...],
                 preferred_element_type=_f32)
    _acc(o_ref, jnp.concatenate([s, ss, jnp.zeros((6, CH), _f32)], axis=0))


def _bn2_stats(g256, xk1024, xq, KP1, A1t, C1t, KP2, pb2t, T2, GM):
    return pl.pallas_call(
        _k2_body,
        grid=(GRID,),
        in_specs=[
            pl.BlockSpec((BR, 256), lambda i: (i, 0)),
            pl.BlockSpec((BR, 1024), lambda i: (i, 0)),
            pl.BlockSpec((BR, CH), lambda i: (i, 0)),
            pl.BlockSpec((256, 256), lambda i: (0, 0)),
            pl.BlockSpec((1, 256), lambda i: (0, 0)),
            pl.BlockSpec((1, 256), lambda i: (0, 0)),
            pl.BlockSpec((256, 1024), lambda i: (0, 0)),
            pl.BlockSpec((1, 1024), lambda i: (0, 0)),
            pl.BlockSpec((CH, 1024), lambda i: (0, 0)),
            pl.BlockSpec((1024, CH), lambda i: (0, 0)),
        ],
        out_specs=pl.BlockSpec((8, CH), lambda i: (0, 0)),
        out_shape=jax.ShapeDtypeStruct((8, CH), _f32),
    )(g256, xk1024, xq, KP1, A1t, C1t, KP2, pb2t, T2, GM)


def _k3_body(g_ref, xk_ref, xq_ref, kp1_ref, a1_ref, c1_ref, kp2_ref, pb2_ref,
             t2_ref, a2_ref, c2_ref, kw1_ref, wb1_ref, f8_ref, h_ref, o_ref):
    w = _w1024(g_ref, xk_ref, xq_ref, kp1_ref, a1_ref, c1_ref, kp2_ref,
               pb2_ref, t2_ref)
    h2 = jnp.maximum(w * a2_ref[...] + c2_ref[...], 0.0)
    hp = jnp.dot(h2, kw1_ref[...], preferred_element_type=_f32) + wb1_ref[...]
    h_ref[...] = hp
    s = jnp.dot(jnp.sum(hp, axis=0, keepdims=True), f8_ref[...],
                preferred_element_type=_f32)
    ss = jnp.dot(jnp.sum(hp * hp, axis=0, keepdims=True), f8_ref[...],
                 preferred_element_type=_f32)
    _acc(o_ref, jnp.concatenate([s, ss, jnp.zeros((6, 8), _f32)], axis=0))


def _k3(g256, xk1024, xq, KP1, A1t, C1t, KP2, pb2t, T2, A2t, C2t, KW1, wb1t, F8):
    return pl.pallas_call(
        _k3_body,
        grid=(GRID,),
        in_specs=[
            pl.BlockSpec((BR, 256), lambda i: (i, 0)),
            pl.BlockSpec((BR, 1024), lambda i: (i, 0)),
            pl.BlockSpec((BR, CH), lambda i: (i, 0)),
            pl.BlockSpec((256, 256), lambda i: (0, 0)),
            pl.BlockSpec((1, 256), lambda i: (0, 0)),
            pl.BlockSpec((1, 256), lambda i: (0, 0)),
            pl.BlockSpec((256, 1024), lambda i: (0, 0)),
            pl.BlockSpec((1, 1024), lambda i: (0, 0)),
            pl.BlockSpec((CH, 1024), lambda i: (0, 0)),
            pl.BlockSpec((1, 1024), lambda i: (0, 0)),
            pl.BlockSpec((1, 1024), lambda i: (0, 0)),
            pl.BlockSpec((1024, 128), lambda i: (0, 0)),
            pl.BlockSpec((1, 128), lambda i: (0, 0)),
            pl.BlockSpec((128, 8), lambda i: (0, 0)),
        ],
        out_specs=[
            pl.BlockSpec((BR, 128), lambda i: (i, 0)),
            pl.BlockSpec((8, 8), lambda i: (0, 0)),
        ],
        out_shape=[
            jax.ShapeDtypeStruct((N, 128), _f32),
            jax.ShapeDtypeStruct((8, 8), _f32),
        ],
    )(g256, xk1024, xq, KP1, A1t, C1t, KP2, pb2t, T2, A2t, C2t, KW1, wb1t, F8)


def _k4_body(h_ref, g_ref, xv_ref, kp1_ref, a1_ref, c1_ref, kp2_ref, pb2_ref,
             a3_ref, c3_ref, wblk_ref, wb2_ref, dm_ref, em_ref, gm_ref, o_ref):
    h3 = jnp.maximum(h_ref[...] * a3_ref[...] + c3_ref[...], 0.0)
    lg = jnp.dot(h3, wblk_ref[...], preferred_element_type=_f32) + wb2_ref[...]
    mx = jnp.max(lg, axis=1, keepdims=True)
    e = jnp.exp(lg - mx)
    den = jnp.dot(e, dm_ref[...], preferred_element_type=_f32)
    sm = e / den
    wt = jnp.dot(sm, em_ref[...], preferred_element_type=_f32)
    pr = _pr1024(g_ref, kp1_ref, a1_ref, c1_ref, kp2_ref, pb2_ref)
    vw = (xv_ref[...].astype(_f32) + pr) * wt
    o_ref[...] = jnp.dot(vw, gm_ref[...], preferred_element_type=_f32)


def _k4(hpre, g256, xv1024, KP1, A1t, C1t, KP2, pb2t, a3t, c3t,
        Wblk, wb2t, DM, EM, GM):
    return pl.pallas_call(
        _k4_body,
        grid=(GRID,),
        in_specs=[
            pl.BlockSpec((BR, 128), lambda i: (i, 0)),
            pl.BlockSpec((BR, 256), lambda i: (i, 0)),
            pl.BlockSpec((BR, 1024), lambda i: (i, 0)),
            pl.BlockSpec((256, 256), lambda i: (0, 0)),
            pl.BlockSpec((1, 256), lambda i: (0, 0)),
            pl.BlockSpec((1, 256), lambda i: (0, 0)),
            pl.BlockSpec((256, 1024), lambda i: (0, 0)),
            pl.BlockSpec((1, 1024), lambda i: (0, 0)),
            pl.BlockSpec((1, 128), lambda i: (0, 0)),
            pl.BlockSpec((1, 128), lambda i: (0, 0)),
            pl.BlockSpec((128, 128), lambda i: (0, 0)),
            pl.BlockSpec((1, 128), lambda i: (0, 0)),
            pl.BlockSpec((128, 128), lambda i: (0, 0)),
            pl.BlockSpec((128, 1024), lambda i: (0, 0)),
            pl.BlockSpec((1024, CH), lambda i: (0, 0)),
        ],
        out_specs=pl.BlockSpec((BR, CH), lambda i: (i, 0)),
        out_shape=jax.ShapeDtypeStruct((N, CH), _f32),
    )(hpre, g256, xv1024, KP1, A1t, C1t, KP2, pb2t, a3t, c3t,
      Wblk, wb2t, DM, EM, GM)


def kernel(p, x, o, edges, boundary, Wq, bq, Wk, bk, Wv, bv, pW1, pb1, pg,
           pbeta, pW2, pb2, wg1, wbeta1, wW1, wb1, wg2, wbeta2, wW2, wb2):
    p_pad = jnp.pad(p, ((0, 0), (0, 13)))
    edges = edges.astype(jnp.int32)
    boundary = boundary.astype(jnp.int32)

    xq, xk, xv = _proj(x, Wq, bq, Wk, bk, Wv, bv)
    idx, gxyz2 = _sc_idx_gxyz(p_pad, edges, boundary)
    xkg2, xvg2 = _sc_kv_gather(xk, xv, idx)
    g256 = gxyz2.reshape(N, 256)
    xk1024 = xkg2.reshape(N, NS * CH)
    xv1024 = xvg2.reshape(N, NS * CH)

    pW1p = jnp.zeros((16, 16), _f32).at[:3, :3].set(pW1)
    pb1p = jnp.zeros((16,), _f32).at[:3].set(pb1)
    pW2p = jnp.zeros((16, CH), _f32).at[:3].set(pW2)
    pgp = jnp.ones((16,), _f32).at[:3].set(pg)
    pbp = jnp.zeros((16,), _f32).at[:3].set(pbeta)

    I8 = jnp.eye(8, dtype=_f32)
    I16 = jnp.eye(16, dtype=_f32)
    I64 = jnp.eye(CH, dtype=_f32)
    KP1 = jnp.kron(I16, pW1p)
    KP2 = jnp.kron(I16, pW2p)
    T2 = jnp.kron(jnp.ones((1, 16), _f32), I64)
    GM = jnp.kron(jnp.ones((16, 1), _f32), I64)
    F8 = jnp.kron(jnp.ones((16, 1), _f32), I8)
    F16 = jnp.kron(jnp.ones((16, 1), _f32), I16)
    KW1 = jnp.kron(I16, wW1)
    Wblk = jnp.kron(I16, wW2)
    DM = jnp.kron(jnp.ones((16, 16), _f32), I8)
    EM = jnp.kron(I16, jnp.kron(jnp.ones((1, 8), _f32), I8))
    pb1t = jnp.tile(pb1p, 16).reshape(1, 256)

    st1 = _bn1_stats(g256, KP1, pb1t, F16)
    m1 = st1[0] / M
    v1 = st1[1] / M - m1 * m1
    a1 = pgp / jnp.sqrt(v1 + EPS)
    c1 = pbp - m1 * a1
    C1 = c1 + a1 * pb1p
    A1t = jnp.tile(a1, 16).reshape(1, 256)
    C1t = jnp.tile(C1, 16).reshape(1, 256)
    pb2t = jnp.tile(pb2, 16).reshape(1, 1024)

    st2 = _bn2_stats(g256, xk1024, xq, KP1, A1t, C1t, KP2, pb2t, T2, GM)
    m2 = st2[0] / M
    v2 = st2[1] / M - m2 * m2
    a2 = wg1 / jnp.sqrt(v2 + EPS)
    c2 = wbeta1 - m2 * a2
    A2t = jnp.tile(a2, 16).reshape(1, 1024)
    C2t = jnp.tile(c2, 16).reshape(1, 1024)

    hpre, st3 = _k3(g256, xk1024, xq, KP1, A1t, C1t, KP2, pb2t, T2,
                    A2t, C2t, KW1, jnp.tile(wb1, 16).reshape(1, 128), F8)
    m3 = st3[0] / M
    v3 = st3[1] / M - m3 * m3
    a3 = wg2 / jnp.sqrt(v3 + EPS)
    c3 = wbeta2 - m3 * a3
    a3t = jnp.tile(a3, 16).reshape(1, 128)
    c3t = jnp.tile(c3, 16).reshape(1, 128)
    wb2t = jnp.tile(wb2, 16).reshape(1, 128)

    return _k4(hpre, g256, xv1024, KP1, A1t, C1t, KP2, pb2t, a3t, c3t,
               Wblk, wb2t, DM, EM, GM)
